# baseline (speedup 1.0000x reference)
import functools

import jax
import jax.numpy as jnp
from jax import lax
from jax.experimental import pallas as pl
from jax.experimental.pallas import tpu as pltpu
from jax.experimental.pallas import tpu_sc as plsc

N_ROWS = 100000
N_UPD = 16384
WIDTH = 128
L = 16
NW = 32
RPW = 3128
M_VECS = (RPW + L - 1) // L
M_PAD = M_VECS * L
GCH = 128
W_CAP = M_PAD + 2 * GCH
CPY = 128
SCAN_VECS = N_UPD // L


def _impl(planes_x, idx32, planes_v):
    np_ = len(planes_x)
    mesh = plsc.VectorSubcoreMesh(
        core_axis_name="c", subcore_axis_name="s", num_cores=2,
        num_subcores=16)

    scratch = [
        pltpu.VMEM((N_UPD,), jnp.int32),
        pltpu.VMEM((M_PAD,), jnp.int32),
        pltpu.VMEM((W_CAP,), jnp.int32),
        pltpu.VMEM((W_CAP,), jnp.int32),
        pltpu.VMEM((GCH,), jnp.int32),
    ]
    for _ in range(np_):
        scratch.append(pltpu.VMEM((GCH, WIDTH), jnp.int32))
        scratch.append(pltpu.VMEM((2, CPY, WIDTH), jnp.int32))
        scratch.append(pltpu.SemaphoreType.DMA((2,)))
        scratch.append(pltpu.SemaphoreType.DMA((2,)))
        scratch.append(pltpu.SemaphoreType.DMA)
        scratch.append(pltpu.SemaphoreType.DMA)

    @functools.partial(
        pl.kernel,
        out_type=tuple(jax.ShapeDtypeStruct((N_ROWS, WIDTH), jnp.int32)
                       for _ in range(np_)),
        mesh=mesh,
        compiler_params=pltpu.CompilerParams(needs_layout_passes=False),
        scratch_types=scratch,
    )
    def run(*refs):
        xs = refs[:np_]
        idx_hbm = refs[np_]
        vs = refs[np_ + 1:2 * np_ + 1]
        outs = refs[2 * np_ + 1:3 * np_ + 1]
        idx_v, m_v, wpos, wrow, sidx = refs[3 * np_ + 1:3 * np_ + 6]
        pp = refs[3 * np_ + 6:]
        gbufs = [pp[6 * p + 0] for p in range(np_)]
        cbs = [pp[6 * p + 1] for p in range(np_)]
        isems = [pp[6 * p + 2] for p in range(np_)]
        osems = [pp[6 * p + 3] for p in range(np_)]
        gsems = [pp[6 * p + 4] for p in range(np_)]
        ssems = [pp[6 * p + 5] for p in range(np_)]

        i32 = jnp.int32
        wid = lax.axis_index("s") * i32(2) + lax.axis_index("c")
        lo = pl.multiple_of(wid * i32(RPW), 8)
        rows_mine = jnp.minimum(i32(RPW), i32(N_ROWS) - lo)
        lanes = lax.iota(jnp.int32, L)

        pltpu.sync_copy(idx_hbm, idx_v)

        def init_m(v, carry):
            m_v[pl.ds(v * i32(L), L)] = jnp.full((L,), -1, jnp.int32)
            return carry
        lax.fori_loop(i32(0), i32(M_VECS), init_m, i32(0))

        def scan_step(v, carry):
            base = v * i32(L)
            ivec = idx_v[pl.ds(base, L)]
            r = ivec - lo
            inm = (r >= 0) & (r < rows_mine)
            cnt = plsc.all_reduce_population_count(inm)[0]
            pos = base + lanes
            rc = jnp.where(inm, r, i32(0))

            @pl.when(cnt == 1)
            def _():
                plsc.store_scatter(m_v, [rc], pos, mask=inm)

            @pl.when(cnt > 1)
            def _():
                rm = jnp.where(inm, r, i32(-1))
                lose = jnp.zeros((L,), jnp.bool_)
                for k in range(1, L):
                    shifted = rm.at[jnp.minimum(lanes + i32(k), i32(L - 1))].get(
                        mode="promise_in_bounds")
                    valid = lanes < i32(L - k)
                    lose = lose | (valid & (shifted == rm))
                keep = inm & jnp.logical_not(lose)
                plsc.store_scatter(m_v, [rc], pos, mask=keep)

            return carry
        lax.fori_loop(i32(0), i32(SCAN_VECS), scan_step, i32(0))

        nt = (rows_mine + i32(CPY - 1)) // i32(CPY)
        last_rel = rows_mine - i32(CPY)

        def chunk_abs(c):
            rel = jnp.minimum(c * i32(CPY), last_rel)
            return pl.multiple_of(lo + rel, 8)

        def in_descs(c, b):
            return [pltpu.make_async_copy(
                xs[p].at[pl.ds(chunk_abs(c), CPY)], cbs[p].at[b],
                isems[p].at[b]) for p in range(np_)]

        def out_descs(c, b):
            return [pltpu.make_async_copy(
                cbs[p].at[b], outs[p].at[pl.ds(chunk_abs(c), CPY)],
                osems[p].at[b]) for p in range(np_)]

        for d in in_descs(i32(0), i32(0)):
            d.start()

        def copy_body(c, carry):
            b = c & i32(1)
            bn = (c + i32(1)) & i32(1)

            @pl.when(c + i32(1) < nt)
            def _():
                @pl.when(c >= i32(1))
                def _():
                    for d in out_descs(c - i32(1), bn):
                        d.wait()
                for d in in_descs(c + i32(1), bn):
                    d.start()

            for d in in_descs(c, b):
                d.wait()
            for d in out_descs(c, b):
                d.start()
            return carry
        lax.fori_loop(i32(0), nt, copy_body, i32(0))

        for d in out_descs(nt - i32(2), nt & i32(1)):
            d.wait()
        for d in out_descs(nt - i32(1), (nt - i32(1)) & i32(1)):
            d.wait()

        def compress_step(v, carry):
            wcount, lastrow, lastpos = carry
            m = m_v[pl.ds(v * i32(L), L)]
            msk = m >= i32(0)
            cnt = plsc.all_reduce_population_count(msk)[0]
            rows = lo + v * i32(L) + lanes
            plsc.store_compressed(wpos.at[pl.ds(wcount, L)], m, mask=msk)
            plsc.store_compressed(wrow.at[pl.ds(wcount, L)], rows, mask=msk)
            lr = jnp.max(jnp.where(msk, rows, i32(-1)))
            lp = jnp.max(jnp.where(msk & (rows == lr), m, i32(-1)))
            lastrow = jnp.where(cnt > i32(0), lr, lastrow)
            lastpos = jnp.where(cnt > i32(0), lp, lastpos)
            return (wcount + cnt, lastrow, lastpos)

        wcount, lastrow, lastpos = lax.fori_loop(
            i32(0), i32(M_VECS), compress_step,
            (jnp.int32(0), jnp.int32(0), jnp.int32(0)))

        @pl.when(wcount > i32(0))
        def _():
            prow = jnp.full((L,), lastrow, jnp.int32)
            ppos = jnp.full((L,), lastpos, jnp.int32)
            for j in range(GCH // L):
                wrow[pl.ds(wcount + i32(j * L), L)] = prow
                wpos[pl.ds(wcount + i32(j * L), L)] = ppos

        nch = (wcount + i32(GCH - 1)) // i32(GCH)

        def win_step(c, carry):
            off = c * i32(GCH)
            gds = [pltpu.make_async_copy(
                vs[p].at[wpos.at[pl.ds(off, GCH)]], gbufs[p], gsems[p])
                for p in range(np_)]
            for d in gds:
                d.start()
            for j in range(GCH // L):
                sidx[pl.ds(j * L, L)] = wrow[pl.ds(off + i32(j * L), L)]
            for d in gds:
                d.wait()
            sds = [pltpu.make_async_copy(
                gbufs[p], outs[p].at[sidx], ssems[p]) for p in range(np_)]
            for d in sds:
                d.start()
            for d in sds:
                d.wait()
            return carry
        lax.fori_loop(i32(0), nch, win_step, i32(0))

    return run(*planes_x, idx32, *planes_v)


def _split64(a, rows_per_block):
    n = a.shape[0]
    grid = (n // rows_per_block,)

    def body(a_ref, lo_ref, hi_ref):
        x = a_ref[...]
        lo_ref[...] = lax.convert_element_type(
            x & jnp.int64(0xFFFFFFFF), jnp.uint32).astype(jnp.int32)
        hi_ref[...] = lax.convert_element_type(
            x >> jnp.int64(32), jnp.int32)

    blk = pl.BlockSpec((rows_per_block, WIDTH), lambda i: (i, 0))
    return pl.pallas_call(
        body, grid=grid, in_specs=[blk], out_specs=[blk, blk],
        out_shape=[jax.ShapeDtypeStruct((n, WIDTH), jnp.int32)] * 2,
    )(a)


def _merge64(lo32, hi32, rows_per_block):
    n = lo32.shape[0]
    grid = (n // rows_per_block,)

    def body(lo_ref, hi_ref, o_ref):
        lo64 = lax.convert_element_type(
            lax.bitcast_convert_type(lo_ref[...], jnp.uint32), jnp.int64)
        hi64 = lax.convert_element_type(hi_ref[...], jnp.int64)
        o_ref[...] = (hi64 << jnp.int64(32)) | lo64

    blk = pl.BlockSpec((rows_per_block, WIDTH), lambda i: (i, 0))
    return pl.pallas_call(
        body, grid=grid, in_specs=[blk, blk], out_specs=blk,
        out_shape=jax.ShapeDtypeStruct((n, WIDTH), jnp.int64),
    )(lo32, hi32)


def kernel(input, index, value):
    idx32 = index.astype(jnp.int32)
    if input.dtype == jnp.int64:
        def split(a):
            at = jnp.transpose(lax.bitcast_convert_type(a, jnp.int32),
                               (2, 0, 1))
            return at[0], at[1]

        xlo, xhi = split(input)
        vlo, vhi = split(value)
        outlo, outhi = _impl([xlo, xhi], idx32, [vlo, vhi])
        ot = jnp.stack([outlo, outhi], axis=0)
        return lax.bitcast_convert_type(
            jnp.transpose(ot, (1, 2, 0)), jnp.int64)
    out, = _impl([input], idx32, [value])
    return out


# speedup vs baseline: 1.0474x; 1.0474x over previous
import functools

import jax
import jax.numpy as jnp
from jax import lax
from jax.experimental import pallas as pl
from jax.experimental.pallas import tpu as pltpu
from jax.experimental.pallas import tpu_sc as plsc

N_ROWS = 100000
N_UPD = 16384
WIDTH = 128
L = 16
NW = 32
RPW = 3128
M_VECS = (RPW + L - 1) // L
M_PAD = M_VECS * L
GCH = 128
W_CAP = M_PAD + 2 * GCH
CPY = 128
SCAN_VECS = N_UPD // L


def _impl(planes_x, idx32, planes_v):
    np_ = len(planes_x)
    mesh = plsc.VectorSubcoreMesh(
        core_axis_name="c", subcore_axis_name="s", num_cores=2,
        num_subcores=16)

    scratch = [
        pltpu.VMEM((N_UPD,), jnp.int32),
        pltpu.VMEM((M_PAD,), jnp.int32),
        pltpu.VMEM((W_CAP,), jnp.int32),
        pltpu.VMEM((W_CAP,), jnp.int32),
        pltpu.VMEM((GCH,), jnp.int32),
    ]
    for _ in range(np_):
        scratch.append(pltpu.VMEM((GCH, WIDTH), jnp.int32))
        scratch.append(pltpu.VMEM((2, CPY, WIDTH), jnp.int32))
        scratch.append(pltpu.SemaphoreType.DMA((2,)))
        scratch.append(pltpu.SemaphoreType.DMA((2,)))
        scratch.append(pltpu.SemaphoreType.DMA)
        scratch.append(pltpu.SemaphoreType.DMA)

    @functools.partial(
        pl.kernel,
        out_type=tuple(jax.ShapeDtypeStruct((N_ROWS, WIDTH), jnp.int32)
                       for _ in range(np_)),
        mesh=mesh,
        compiler_params=pltpu.CompilerParams(needs_layout_passes=False),
        scratch_types=scratch,
    )
    def run(*refs):
        xs = refs[:np_]
        idx_hbm = refs[np_]
        vs = refs[np_ + 1:2 * np_ + 1]
        outs = refs[2 * np_ + 1:3 * np_ + 1]
        idx_v, m_v, wpos, wrow, sidx = refs[3 * np_ + 1:3 * np_ + 6]
        pp = refs[3 * np_ + 6:]
        gbufs = [pp[6 * p + 0] for p in range(np_)]
        cbs = [pp[6 * p + 1] for p in range(np_)]
        isems = [pp[6 * p + 2] for p in range(np_)]
        osems = [pp[6 * p + 3] for p in range(np_)]
        gsems = [pp[6 * p + 4] for p in range(np_)]
        ssems = [pp[6 * p + 5] for p in range(np_)]

        i32 = jnp.int32
        wid = lax.axis_index("s") * i32(2) + lax.axis_index("c")
        lo = pl.multiple_of(wid * i32(RPW), 8)
        rows_mine = jnp.minimum(i32(RPW), i32(N_ROWS) - lo)
        lanes = lax.iota(jnp.int32, L)

        pltpu.sync_copy(idx_hbm, idx_v)

        def init_m(v, carry):
            m_v[pl.ds(v * i32(L), L)] = jnp.full((L,), -1, jnp.int32)
            return carry
        lax.fori_loop(i32(0), i32(M_VECS), init_m, i32(0))

        def scan_step(v, carry):
            base = v * i32(L)
            ivec = idx_v[pl.ds(base, L)]
            r = ivec - lo
            inm = (r >= 0) & (r < rows_mine)
            cnt = plsc.all_reduce_population_count(inm)[0]
            pos = base + lanes
            rc = jnp.where(inm, r, i32(0))

            @pl.when(cnt == 1)
            def _():
                plsc.store_scatter(m_v, [rc], pos, mask=inm)

            @pl.when(cnt > 1)
            def _():
                rm = jnp.where(inm, r, i32(-1))
                lose = jnp.zeros((L,), jnp.bool_)
                for k in range(1, L):
                    shifted = rm.at[jnp.minimum(lanes + i32(k), i32(L - 1))].get(
                        mode="promise_in_bounds")
                    valid = lanes < i32(L - k)
                    lose = lose | (valid & (shifted == rm))
                keep = inm & jnp.logical_not(lose)
                plsc.store_scatter(m_v, [rc], pos, mask=keep)

            return carry

        def scan_range(a, b_):
            lax.fori_loop(a, b_, scan_step, i32(0))

        nt = (rows_mine + i32(CPY - 1)) // i32(CPY)
        last_rel = rows_mine - i32(CPY)

        def chunk_abs(c):
            rel = jnp.minimum(c * i32(CPY), last_rel)
            return pl.multiple_of(lo + rel, 8)

        def in_descs(c, b):
            return [pltpu.make_async_copy(
                xs[p].at[pl.ds(chunk_abs(c), CPY)], cbs[p].at[b],
                isems[p].at[b]) for p in range(np_)]

        def out_descs(c, b):
            return [pltpu.make_async_copy(
                cbs[p].at[b], outs[p].at[pl.ds(chunk_abs(c), CPY)],
                osems[p].at[b]) for p in range(np_)]

        for d in in_descs(i32(0), i32(0)):
            d.start()

        seg = (SCAN_VECS + 23) // 24

        def copy_body(c, carry):
            b = c & i32(1)
            bn = (c + i32(1)) & i32(1)

            @pl.when(c + i32(1) < nt)
            def _():
                @pl.when(c >= i32(1))
                def _():
                    for d in out_descs(c - i32(1), bn):
                        d.wait()
                for d in in_descs(c + i32(1), bn):
                    d.start()

            scan_range(jnp.minimum(c * i32(seg), i32(SCAN_VECS)),
                       jnp.minimum((c + i32(1)) * i32(seg), i32(SCAN_VECS)))

            for d in in_descs(c, b):
                d.wait()
            for d in out_descs(c, b):
                d.start()
            return carry
        lax.fori_loop(i32(0), nt, copy_body, i32(0))
        scan_range(jnp.minimum(nt * i32(seg), i32(SCAN_VECS)), i32(SCAN_VECS))

        for d in out_descs(nt - i32(2), nt & i32(1)):
            d.wait()
        for d in out_descs(nt - i32(1), (nt - i32(1)) & i32(1)):
            d.wait()

        def compress_step(v, carry):
            wcount, lastrow, lastpos = carry
            m = m_v[pl.ds(v * i32(L), L)]
            msk = m >= i32(0)
            cnt = plsc.all_reduce_population_count(msk)[0]
            rows = lo + v * i32(L) + lanes
            plsc.store_compressed(wpos.at[pl.ds(wcount, L)], m, mask=msk)
            plsc.store_compressed(wrow.at[pl.ds(wcount, L)], rows, mask=msk)
            lr = jnp.max(jnp.where(msk, rows, i32(-1)))
            lp = jnp.max(jnp.where(msk & (rows == lr), m, i32(-1)))
            lastrow = jnp.where(cnt > i32(0), lr, lastrow)
            lastpos = jnp.where(cnt > i32(0), lp, lastpos)
            return (wcount + cnt, lastrow, lastpos)

        wcount, lastrow, lastpos = lax.fori_loop(
            i32(0), i32(M_VECS), compress_step,
            (jnp.int32(0), jnp.int32(0), jnp.int32(0)))

        @pl.when(wcount > i32(0))
        def _():
            prow = jnp.full((L,), lastrow, jnp.int32)
            ppos = jnp.full((L,), lastpos, jnp.int32)
            for j in range(GCH // L):
                wrow[pl.ds(wcount + i32(j * L), L)] = prow
                wpos[pl.ds(wcount + i32(j * L), L)] = ppos

        nch = (wcount + i32(GCH - 1)) // i32(GCH)

        def win_step(c, carry):
            off = c * i32(GCH)
            gds = [pltpu.make_async_copy(
                vs[p].at[wpos.at[pl.ds(off, GCH)]], gbufs[p], gsems[p])
                for p in range(np_)]
            for d in gds:
                d.start()
            for j in range(GCH // L):
                sidx[pl.ds(j * L, L)] = wrow[pl.ds(off + i32(j * L), L)]
            for d in gds:
                d.wait()
            sds = [pltpu.make_async_copy(
                gbufs[p], outs[p].at[sidx], ssems[p]) for p in range(np_)]
            for d in sds:
                d.start()
            for d in sds:
                d.wait()
            return carry
        lax.fori_loop(i32(0), nch, win_step, i32(0))

    return run(*planes_x, idx32, *planes_v)


def kernel(input, index, value):
    idx32 = index.astype(jnp.int32)
    if input.dtype == jnp.int64:
        mask = jnp.int64(0xFFFFFFFF)

        def split(a):
            lo_p = lax.bitcast_convert_type(
                lax.convert_element_type(a & mask, jnp.uint32), jnp.int32)
            hi_p = lax.convert_element_type(a >> jnp.int64(32), jnp.int32)
            return lo_p, hi_p

        xlo, xhi = split(input)
        vlo, vhi = split(value)
        outlo, outhi = _impl([xlo, xhi], idx32, [vlo, vhi])
        lo64 = lax.convert_element_type(
            lax.bitcast_convert_type(outlo, jnp.uint32), jnp.int64)
        hi64 = lax.convert_element_type(outhi, jnp.int64) << jnp.int64(32)
        return hi64 | lo64
    out, = _impl([input], idx32, [value])
    return out
